# baseline (speedup 1.0000x reference)
import functools
import math

import jax
import jax.numpy as jnp
from jax import lax
from jax.experimental import pallas as pl
from jax.experimental.pallas import tpu as pltpu
from jax.experimental.pallas import tpu_sc as plsc

D_MODEL = 64
SCALE = math.sqrt(D_MODEL)
CH = 128


def _make_kernel(V, D, B, NC, NS, L):
    NW = NC * NS
    b_per_w = B // NW
    n_chunks = b_per_w // CH
    mesh = plsc.VectorSubcoreMesh(core_axis_name="c", subcore_axis_name="s")

    @functools.partial(
        pl.kernel,
        mesh=mesh,
        compiler_params=pltpu.CompilerParams(use_tc_tiling_on_sc=False),
        out_type=jax.ShapeDtypeStruct((B, D), jnp.float32),
        scratch_types=[
            pltpu.VMEM((n_chunks, CH), jnp.int32),
            pltpu.VMEM((b_per_w, D), jnp.float32),
            pltpu.SemaphoreType.DMA,
        ],
    )
    def emb_kernel(idx_hbm, table_hbm, out_hbm, idx_v, rows_v, sem):
        wid = lax.axis_index("s") * NC + lax.axis_index("c")
        base = wid * b_per_w
        pltpu.sync_copy(idx_hbm.at[pl.ds(wid * n_chunks, n_chunks)], idx_v)
        copies = []
        for c in range(n_chunks):
            copies.append(
                pltpu.async_copy(
                    table_hbm.at[idx_v.at[c]],
                    rows_v.at[pl.ds(c * CH, CH)],
                    sem,
                )
            )
        for cp in copies:
            cp.wait()

        rows_per_iter = 8
        def body(i, _):
            r0 = i * rows_per_iter
            for r in range(rows_per_iter):
                for j in range(D // L):
                    sl = pl.ds(j * L, L)
                    rows_v[r0 + r, sl] = rows_v[r0 + r, sl] * SCALE
            return ()
        lax.fori_loop(0, b_per_w // rows_per_iter, body, ())

        pltpu.sync_copy(rows_v, out_hbm.at[pl.ds(base, b_per_w)])

    return emb_kernel


def kernel(tokens, embedding):
    seq_len, batch = tokens.shape
    V, D = embedding.shape
    B = seq_len * batch
    info = plsc.get_sparse_core_info()
    NC, NS, L = info.num_cores, info.num_subcores, info.num_lanes
    idx = tokens.reshape(B // CH, CH).astype(jnp.int32)
    emb_kernel = _make_kernel(V, D, B, NC, NS, L)
    out = emb_kernel(idx, embedding)
    return out.reshape(seq_len, batch, D)


# speedup vs baseline: 1.5564x; 1.5564x over previous
import functools
import math

import jax
import jax.numpy as jnp
from jax import lax
from jax.experimental import pallas as pl
from jax.experimental.pallas import tpu as pltpu
from jax.experimental.pallas import tpu_sc as plsc


def _make_kernel(V, D, S, B, NC, NS, L):
    NW = NC * NS
    T = S * B
    t_per_w = T // NW
    C = 64
    n_chunks = t_per_w // C
    scale = math.sqrt(D)
    mesh = plsc.VectorSubcoreMesh(core_axis_name="c", subcore_axis_name="s")

    @functools.partial(
        pl.kernel,
        mesh=mesh,
        compiler_params=pltpu.CompilerParams(use_tc_tiling_on_sc=True),
        out_type=jax.ShapeDtypeStruct((T // 2, 2 * D), jnp.float32),
        scratch_types=[
            pltpu.VMEM((t_per_w,), jnp.int32),
            pltpu.VMEM((C * 8, D), jnp.float32),
            pltpu.VMEM((t_per_w // 2, 2 * D), jnp.float32),
            pltpu.SemaphoreType.DMA,
        ],
    )
    def emb_kernel(idx_hbm, table_hbm, out_hbm, idx_v, stage, tokbuf, sem):
        wid = lax.axis_index("c") * NS + lax.axis_index("s")
        base = pl.multiple_of(wid * t_per_w, t_per_w)
        obase = pl.multiple_of(wid * (t_per_w // 2), t_per_w // 2)

        pltpu.sync_copy(idx_hbm.at[pl.ds(base, t_per_w)], idx_v)

        def chunk_body(c, _):
            c0 = c * C

            def fire(g, _):
                vec = idx_v[pl.ds(c0 + g * L, L)]
                for k in range(L):
                    v = vec[k]
                    g8 = pl.multiple_of(
                        lax.shift_right_logical(v, 3) * 8, 8
                    )
                    pltpu.async_copy(
                        table_hbm.at[pl.ds(g8, 8), :],
                        stage.at[pl.ds((g * L + k) * 8, 8), :],
                        sem,
                    )
                return ()

            lax.fori_loop(0, C // L, fire, ())
            pltpu.make_async_copy(
                table_hbm.at[pl.ds(0, C * 8), :], stage, sem
            ).wait()

            def extract(g, _):
                t0 = c0 + g * L
                vec = idx_v[pl.ds(t0, L)]
                for k in range(L):
                    v = vec[k]
                    r = (g * L + k) * 8 + (v & 7)
                    row = t0 // 2 + k // 2
                    col = (k % 2) * D
                    for j in range(D // L):
                        tokbuf[row, pl.ds(col + j * L, L)] = (
                            stage[r, pl.ds(j * L, L)] * scale
                        )
                return ()

            lax.fori_loop(0, C // L, extract, ())
            return ()

        lax.fori_loop(0, n_chunks, chunk_body, ())

        pltpu.sync_copy(tokbuf, out_hbm.at[pl.ds(obase, t_per_w // 2)])

    return emb_kernel


def kernel(tokens, embedding):
    S, B = tokens.shape
    V, D = embedding.shape
    info = plsc.get_sparse_core_info()
    NC, NS, L = info.num_cores, info.num_subcores, info.num_lanes
    idx = tokens.T.reshape(S * B).astype(jnp.int32)
    emb_kernel = _make_kernel(V, D, S, B, NC, NS, L)
    out = emb_kernel(idx, embedding)
    return out.reshape(B, S, D).transpose(1, 0, 2)


# speedup vs baseline: 1.5853x; 1.0186x over previous
import functools
import math

import jax
import jax.numpy as jnp
from jax import lax
from jax.experimental import pallas as pl
from jax.experimental.pallas import tpu as pltpu
from jax.experimental.pallas import tpu_sc as plsc


def _make_kernel(V, D, S, B, NC, NS, L):
    NW = NC * NS
    T = S * B
    t_per_w = T // NW
    C = 32
    n_chunks = t_per_w // C
    scale = math.sqrt(D)
    mesh = plsc.VectorSubcoreMesh(core_axis_name="c", subcore_axis_name="s")

    @functools.partial(
        pl.kernel,
        mesh=mesh,
        compiler_params=pltpu.CompilerParams(use_tc_tiling_on_sc=True),
        out_type=jax.ShapeDtypeStruct((T // 2, 2 * D), jnp.float32),
        scratch_types=[
            pltpu.VMEM((t_per_w,), jnp.int32),
            pltpu.VMEM((C * 8, D), jnp.float32),
            pltpu.VMEM((C * 8, D), jnp.float32),
            pltpu.VMEM((t_per_w // 2, 2 * D), jnp.float32),
            pltpu.SemaphoreType.DMA,
            pltpu.SemaphoreType.DMA,
        ],
    )
    def emb_kernel(
        idx_hbm, table_hbm, out_hbm, idx_v, stage_a, stage_b, tokbuf,
        sem_a, sem_b,
    ):
        wid = lax.axis_index("c") * NS + lax.axis_index("s")
        base = pl.multiple_of(wid * t_per_w, t_per_w)
        obase = pl.multiple_of(wid * (t_per_w // 2), t_per_w // 2)

        pltpu.sync_copy(idx_hbm.at[pl.ds(base, t_per_w)], idx_v)

        def fire(c, stage, sem):
            c0 = c * C

            def body(g, _):
                vec = idx_v[pl.ds(c0 + g * L, L)]
                for k in range(L):
                    v = vec[k]
                    g8 = pl.multiple_of(
                        lax.shift_right_logical(v, 3) * 8, 8
                    )
                    pltpu.async_copy(
                        table_hbm.at[pl.ds(g8, 8), :],
                        stage.at[pl.ds((g * L + k) * 8, 8), :],
                        sem,
                    )
                return ()

            lax.fori_loop(0, C // L, body, ())

        def drain_extract(c, stage, sem):
            pltpu.make_async_copy(
                table_hbm.at[pl.ds(0, C * 8), :], stage, sem
            ).wait()
            c0 = c * C

            def body(g, _):
                t0 = c0 + g * L
                vec = idx_v[pl.ds(t0, L)]
                for k in range(L):
                    v = vec[k]
                    r = (g * L + k) * 8 + (v & 7)
                    row = t0 // 2 + k // 2
                    col = (k % 2) * D
                    for j in range(D // L):
                        tokbuf[row, pl.ds(col + j * L, L)] = (
                            stage[r, pl.ds(j * L, L)] * scale
                        )
                return ()

            lax.fori_loop(0, C // L, body, ())

        fire(0, stage_a, sem_a)

        def pair_body(p, _):
            c = p * 2
            fire(c + 1, stage_b, sem_b)
            drain_extract(c, stage_a, sem_a)

            @pl.when(p < n_chunks // 2 - 1)
            def _():
                fire(c + 2, stage_a, sem_a)

            drain_extract(c + 1, stage_b, sem_b)
            return ()

        lax.fori_loop(0, n_chunks // 2, pair_body, ())

        pltpu.sync_copy(tokbuf, out_hbm.at[pl.ds(obase, t_per_w // 2)])

    return emb_kernel


def kernel(tokens, embedding):
    S, B = tokens.shape
    V, D = embedding.shape
    info = plsc.get_sparse_core_info()
    NC, NS, L = info.num_cores, info.num_subcores, info.num_lanes
    idx = tokens.T.reshape(S * B).astype(jnp.int32)
    emb_kernel = _make_kernel(V, D, S, B, NC, NS, L)
    out = emb_kernel(idx, embedding)
    return out.reshape(B, S, D).transpose(1, 0, 2)
